# Initial kernel scaffold; baseline (speedup 1.0000x reference)
import jax
import jax.numpy as jnp
from jax.experimental import pallas as pl


def kernel(dense_x, sparse_indices, tables, bW0, bb0, bW1, bb1, bW2, bb2, tW0, tb0, tW1, tb1, tW2, tb2):
    raise NotImplementedError("write your pallas kernel here")


# speedup vs baseline: 2.2921x; 2.2921x over previous
import functools

import jax
import jax.numpy as jnp
from jax import lax
from jax.experimental import pallas as pl
from jax.experimental.pallas import tpu as pltpu
from jax.experimental.pallas import tpu_sc as plsc

N_FIELDS = 26
VOCAB = 100000
EMBED_DIM = 32
DENSE_DIM = 13
BATCH = 4096
NV = N_FIELDS + 1
N_PAIRS = (NV * (NV - 1)) // 2


def _make_sc_gather(B, D):
    info = plsc.get_sparse_core_info()
    NC, NS = info.num_cores, info.num_subcores
    NW = NC * NS
    assert B % (8 * NW) == 0
    b_per_w = B // NW
    mesh = plsc.VectorSubcoreMesh(core_axis_name="c", subcore_axis_name="s")

    @functools.partial(
        pl.kernel,
        mesh=mesh,
        out_type=jax.ShapeDtypeStruct((B, D), jnp.float32),
        scratch_types=[
            pltpu.VMEM((b_per_w,), jnp.int32),
            pltpu.VMEM((b_per_w, D), jnp.float32),
            pltpu.SemaphoreType.DMA,
        ],
        compiler_params=pltpu.CompilerParams(use_tc_tiling_on_sc=False),
    )
    def gather_k(table_hbm, idx_hbm, out_hbm, idx_v, rows_v, sem):
        wid = lax.axis_index("s") * NC + lax.axis_index("c")
        base = wid * b_per_w
        pltpu.sync_copy(idx_hbm.at[pl.ds(base, b_per_w)], idx_v)
        pltpu.async_copy(table_hbm.at[idx_v], rows_v, sem).wait()
        pltpu.sync_copy(rows_v, out_hbm.at[pl.ds(base, b_per_w)])

    return gather_k


_sc_gather = _make_sc_gather(BATCH * N_FIELDS, EMBED_DIM)


def _dense_body(x_ref, emb_ref, bW0, bb0, bW1, bb1, bW2, bb2,
                tW0t, tb0c, tW1t, tb1c, tW2t, tb2c, out_ref):
    x = x_ref[...]
    h = jnp.maximum(jnp.dot(x, bW0[...], preferred_element_type=jnp.float32) + bb0[...], 0.0)
    h = jnp.maximum(jnp.dot(h, bW1[...], preferred_element_type=jnp.float32) + bb1[...], 0.0)
    h = jnp.maximum(jnp.dot(h, bW2[...], preferred_element_type=jnp.float32) + bb2[...], 0.0)
    feats = jnp.concatenate([h, emb_ref[...]], axis=1)
    ft = feats.T
    f3 = ft.reshape(NV, EMBED_DIM, ft.shape[1])
    parts = []
    for i in range(1, NV):
        parts.append(jnp.sum(f3[:i] * f3[i][None], axis=1))
    inter_t = jnp.concatenate(parts, axis=0)
    top_t = jnp.concatenate([ft[:EMBED_DIM], inter_t], axis=0)
    t = jnp.maximum(jnp.dot(tW0t[...], top_t, preferred_element_type=jnp.float32) + tb0c[...], 0.0)
    t = jnp.maximum(jnp.dot(tW1t[...], t, preferred_element_type=jnp.float32) + tb1c[...], 0.0)
    o = jnp.dot(tW2t[...], t, preferred_element_type=jnp.float32) + tb2c[...]
    out_ref[...] = 1.0 / (1.0 + jnp.exp(-o))


def _dense_call(x, emb2, bW0, bb0, bW1, bb1, bW2, bb2,
                tW0t, tb0c, tW1t, tb1c, tW2t, tb2c):
    Bblk = 512
    grid = (BATCH // Bblk,)
    full = lambda a: pl.BlockSpec(a.shape, lambda i: (0,) * a.ndim)
    ws = [bW0, bb0, bW1, bb1, bW2, bb2, tW0t, tb0c, tW1t, tb1c, tW2t, tb2c]
    out = pl.pallas_call(
        _dense_body,
        grid=grid,
        in_specs=[
            pl.BlockSpec((Bblk, DENSE_DIM), lambda i: (i, 0)),
            pl.BlockSpec((Bblk, N_FIELDS * EMBED_DIM), lambda i: (i, 0)),
        ] + [full(w) for w in ws],
        out_specs=pl.BlockSpec((1, Bblk), lambda i: (0, i)),
        out_shape=jax.ShapeDtypeStruct((1, BATCH), jnp.float32),
    )(x, emb2, *ws)
    return out.reshape(BATCH, 1)


def kernel(dense_x, sparse_indices, tables, bW0, bb0, bW1, bb1, bW2, bb2,
           tW0, tb0, tW1, tb1, tW2, tb2):
    idx = sparse_indices.astype(jnp.int32)
    flat_idx = (idx + jnp.arange(N_FIELDS, dtype=jnp.int32)[None, :] * VOCAB).reshape(-1)
    tab_flat = tables.reshape(N_FIELDS * VOCAB, EMBED_DIM)
    emb = _sc_gather(tab_flat, flat_idx)
    emb2 = emb.reshape(BATCH, N_FIELDS * EMBED_DIM)
    return _dense_call(
        dense_x, emb2,
        bW0, bb0.reshape(1, -1), bW1, bb1.reshape(1, -1), bW2, bb2.reshape(1, -1),
        tW0.T, tb0.reshape(-1, 1), tW1.T, tb1.reshape(-1, 1), tW2.T, tb2.reshape(-1, 1),
    )
